# baseline (speedup 1.0000x reference)
import functools

import jax
import jax.numpy as jnp
from jax import lax
from jax.experimental import pallas as pl
from jax.experimental.pallas import tpu as pltpu
from jax.experimental.pallas import tpu_sc as plsc

VOCAB = 100000
ENC_DIM = 128
OUT_DIM = 256

_ROWS_PER_BLOCK = 4000


def _proj_body(emb_ref, w_ref, out_ref):
    out_ref[...] = lax.dot_general(
        emb_ref[...], w_ref[...],
        dimension_numbers=(((1,), (1,)), ((), ())),
        preferred_element_type=jnp.float32,
    )


def _project_table(emb_table, w_out):
    n_blocks = VOCAB // _ROWS_PER_BLOCK
    return pl.pallas_call(
        _proj_body,
        grid=(n_blocks,),
        in_specs=[
            pl.BlockSpec((_ROWS_PER_BLOCK, ENC_DIM), lambda i: (i, 0)),
            pl.BlockSpec((OUT_DIM, ENC_DIM), lambda i: (0, 0)),
        ],
        out_specs=pl.BlockSpec((_ROWS_PER_BLOCK, OUT_DIM), lambda i: (i, 0)),
        out_shape=jax.ShapeDtypeStruct((VOCAB, OUT_DIM), jnp.float32),
    )(emb_table, w_out)


_CHUNK = 128


def _make_sc_gather(total_rows):
    info = plsc.get_sparse_core_info()
    nc, ns = info.num_cores, info.num_subcores
    nw = nc * ns
    assert total_rows % (nw * _CHUNK) == 0
    rows_per_w = total_rows // nw
    chunks_per_w = rows_per_w // _CHUNK

    mesh = plsc.VectorSubcoreMesh(core_axis_name="c", subcore_axis_name="s")

    _NBUF = 3
    n_main = chunks_per_w - (chunks_per_w % _NBUF)

    @functools.partial(
        pl.kernel,
        mesh=mesh,
        out_type=jax.ShapeDtypeStruct((total_rows, OUT_DIM), jnp.float32),
        scratch_types=[
            pltpu.VMEM((rows_per_w,), jnp.int32),
            [pltpu.VMEM((_CHUNK, OUT_DIM), jnp.float32)] * _NBUF,
            [pltpu.SemaphoreType.DMA] * _NBUF,
            [pltpu.SemaphoreType.DMA] * _NBUF,
        ],
    )
    def sc_gather(p_hbm, idx_hbm, out_hbm, idx_v, bufs, gsems, wsems):
        wid = lax.axis_index("s") * nc + lax.axis_index("c")
        base = wid * rows_per_w
        pltpu.sync_copy(idx_hbm.at[pl.ds(base, rows_per_w)], idx_v)

        def gather(g, b):
            pltpu.async_copy(
                p_hbm.at[idx_v.at[pl.ds(g * _CHUNK, _CHUNK)]], bufs[b], gsems[b]
            )

        def wait_gather(g, b):
            pltpu.make_async_copy(
                p_hbm.at[idx_v.at[pl.ds(g * _CHUNK, _CHUNK)]], bufs[b], gsems[b]
            ).wait()

        def write(g, b):
            pltpu.async_copy(
                bufs[b], out_hbm.at[pl.ds(base + g * _CHUNK, _CHUNK)], wsems[b]
            )

        def wait_write(g, b):
            pltpu.make_async_copy(
                bufs[b], out_hbm.at[pl.ds(base + g * _CHUNK, _CHUNK)], wsems[b]
            ).wait()

        for b in range(_NBUF - 1):
            gather(b, b)

        def body(t, carry):
            for b in range(_NBUF):
                g = _NBUF * t + b

                @pl.when(g >= 1)
                def _():
                    wait_write(g - 1, (b - 1) % _NBUF)

                @pl.when(g + _NBUF - 1 < chunks_per_w)
                def _():
                    gather(g + _NBUF - 1, (b - 1) % _NBUF)

                wait_gather(g, b)
                write(g, b)
            return carry

        lax.fori_loop(0, n_main // _NBUF, body, 0)
        for g in range(n_main, chunks_per_w):
            b = g % _NBUF
            wait_write(g - 1, (b - 1) % _NBUF)
            wait_gather(g, b)
            write(g, b)
        wait_write(chunks_per_w - 1, (chunks_per_w - 1) % _NBUF)

    return sc_gather, nw, chunks_per_w


def kernel(operation_ids, emb_table, W_out):
    b, n = operation_ids.shape
    total = b * n
    proj = _project_table(emb_table, W_out)
    sc_gather, nw, chunks_per_w = _make_sc_gather(total)
    out_flat = sc_gather(proj, operation_ids.reshape(total))
    return out_flat.reshape(b, n, OUT_DIM)


# speedup vs baseline: 1.0063x; 1.0063x over previous
import functools

import jax
import jax.numpy as jnp
from jax import lax
from jax.experimental import pallas as pl
from jax.experimental.pallas import tpu as pltpu
from jax.experimental.pallas import tpu_sc as plsc

VOCAB = 100000
ENC_DIM = 128
OUT_DIM = 256

_ROWS_PER_BLOCK = 10000


def _proj_body(emb_ref, w_ref, out_ref):
    out_ref[...] = lax.dot_general(
        emb_ref[...], w_ref[...],
        dimension_numbers=(((1,), (1,)), ((), ())),
        preferred_element_type=jnp.float32,
    )


def _project_table(emb_table, w_out):
    n_blocks = VOCAB // _ROWS_PER_BLOCK
    return pl.pallas_call(
        _proj_body,
        grid=(n_blocks,),
        in_specs=[
            pl.BlockSpec((_ROWS_PER_BLOCK, ENC_DIM), lambda i: (i, 0)),
            pl.BlockSpec((OUT_DIM, ENC_DIM), lambda i: (0, 0)),
        ],
        out_specs=pl.BlockSpec((_ROWS_PER_BLOCK, OUT_DIM), lambda i: (i, 0)),
        out_shape=jax.ShapeDtypeStruct((VOCAB, OUT_DIM), jnp.float32),
    )(emb_table, w_out)


_CHUNK = 128


def _make_sc_gather(total_rows):
    info = plsc.get_sparse_core_info()
    nc, ns = info.num_cores, info.num_subcores
    nw = nc * ns
    assert total_rows % (nw * _CHUNK) == 0
    rows_per_w = total_rows // nw
    chunks_per_w = rows_per_w // _CHUNK

    mesh = plsc.VectorSubcoreMesh(core_axis_name="c", subcore_axis_name="s")

    _NBUF = 3
    n_main = chunks_per_w - (chunks_per_w % _NBUF)

    @functools.partial(
        pl.kernel,
        mesh=mesh,
        out_type=jax.ShapeDtypeStruct((total_rows, OUT_DIM), jnp.float32),
        scratch_types=[
            pltpu.VMEM((rows_per_w,), jnp.int32),
            [pltpu.VMEM((_CHUNK, OUT_DIM), jnp.float32)] * _NBUF,
            [pltpu.SemaphoreType.DMA] * _NBUF,
            [pltpu.SemaphoreType.DMA] * _NBUF,
        ],
    )
    def sc_gather(p_hbm, idx_hbm, out_hbm, idx_v, bufs, gsems, wsems):
        wid = lax.axis_index("s") * nc + lax.axis_index("c")
        base = wid * rows_per_w
        pltpu.sync_copy(idx_hbm.at[pl.ds(base, rows_per_w)], idx_v)

        def gather(g, b):
            pltpu.async_copy(
                p_hbm.at[idx_v.at[pl.ds(g * _CHUNK, _CHUNK)]], bufs[b], gsems[b]
            )

        def wait_gather(g, b):
            pltpu.make_async_copy(
                p_hbm.at[idx_v.at[pl.ds(g * _CHUNK, _CHUNK)]], bufs[b], gsems[b]
            ).wait()

        def write(g, b):
            pltpu.async_copy(
                bufs[b], out_hbm.at[pl.ds(base + g * _CHUNK, _CHUNK)], wsems[b]
            )

        def wait_write(g, b):
            pltpu.make_async_copy(
                bufs[b], out_hbm.at[pl.ds(base + g * _CHUNK, _CHUNK)], wsems[b]
            ).wait()

        for b in range(_NBUF - 1):
            gather(b, b)

        def body(t, carry):
            for b in range(_NBUF):
                g = _NBUF * t + b

                @pl.when(g >= 1)
                def _():
                    wait_write(g - 1, (b - 1) % _NBUF)

                @pl.when(g + _NBUF - 1 < chunks_per_w)
                def _():
                    gather(g + _NBUF - 1, (b - 1) % _NBUF)

                wait_gather(g, b)
                write(g, b)
            return carry

        lax.fori_loop(0, n_main // _NBUF, body, 0)
        for g in range(n_main, chunks_per_w):
            b = g % _NBUF
            wait_write(g - 1, (b - 1) % _NBUF)
            wait_gather(g, b)
            write(g, b)
        wait_write(chunks_per_w - 1, (chunks_per_w - 1) % _NBUF)

    return sc_gather, nw, chunks_per_w


def kernel(operation_ids, emb_table, W_out):
    b, n = operation_ids.shape
    total = b * n
    proj = _project_table(emb_table, W_out)
    sc_gather, nw, chunks_per_w = _make_sc_gather(total)
    out_flat = sc_gather(proj, operation_ids.reshape(total))
    return out_flat.reshape(b, n, OUT_DIM)
